# baseline (speedup 1.0000x reference)
import functools

import jax
import jax.numpy as jnp
from jax import lax
from jax.experimental import pallas as pl
from jax.experimental.pallas import tpu as pltpu
from jax.experimental.pallas import tpu_sc as plsc

B = 4096
S = 50
S_PAD = 56
E = 64
NW = 32

ITEMS_PER_TILE = (B * S) // NW
ITEM_CHUNK = 640
N_ITEM_CHUNKS = ITEMS_PER_TILE // ITEM_CHUNK
B_PER_TILE = B // NW
B_GROUP = 8
N_B_GROUPS = B_PER_TILE // B_GROUP
CTX_CHUNK = B_GROUP * S_PAD


def _sc_gather_pool(item_table, ctx_table, item_idx_flat, ctx_idx_flat):
    mesh = plsc.VectorSubcoreMesh(core_axis_name="c", subcore_axis_name="s")

    @functools.partial(
        pl.kernel,
        mesh=mesh,
        out_type=[
            jax.ShapeDtypeStruct((B * S, E), jnp.float32),
            jax.ShapeDtypeStruct((B, E), jnp.float32),
        ],
        scratch_types=[
            pltpu.VMEM((ITEM_CHUNK,), jnp.int32),
            pltpu.VMEM((ITEM_CHUNK, E), jnp.float32),
            pltpu.VMEM((CTX_CHUNK,), jnp.int32),
            pltpu.VMEM((CTX_CHUNK, E), jnp.float32),
            pltpu.VMEM((B_GROUP, E), jnp.float32),
            pltpu.SemaphoreType.DMA,
        ],
        compiler_params=pltpu.CompilerParams(use_tc_tiling_on_sc=False),
    )
    def k(item_table_hbm, ctx_table_hbm, iidx_hbm, cidx_hbm,
          item_out, pooled_out, iidx_v, irows_v, cidx_v, crows_v, pool_v, sem):
        wid = lax.axis_index("s") * 2 + lax.axis_index("c")

        def item_chunk(kk, carry):
            base = pl.multiple_of(wid * ITEMS_PER_TILE + kk * ITEM_CHUNK,
                                  ITEM_CHUNK)
            pltpu.sync_copy(iidx_hbm.at[pl.ds(base, ITEM_CHUNK)], iidx_v)
            pltpu.async_copy(item_table_hbm.at[iidx_v], irows_v, sem).wait()
            pltpu.sync_copy(irows_v, item_out.at[pl.ds(base, ITEM_CHUNK)])
            return carry

        lax.fori_loop(0, N_ITEM_CHUNKS, item_chunk, 0)

        def ctx_group(g, carry):
            cbase = pl.multiple_of(
                wid * (B_PER_TILE * S_PAD) + g * CTX_CHUNK, CTX_CHUNK)
            pltpu.sync_copy(cidx_hbm.at[pl.ds(cbase, CTX_CHUNK)], cidx_v)
            pltpu.async_copy(ctx_table_hbm.at[cidx_v], crows_v, sem).wait()
            zero = jnp.zeros((16,), jnp.float32)
            for bi in range(B_GROUP):
                def body(s, acc, _bi=bi):
                    return tuple(
                        acc[c] + crows_v[_bi * S_PAD + s, pl.ds(c * 16, 16)]
                        for c in range(4))

                acc = lax.fori_loop(0, S_PAD, body, (zero, zero, zero, zero))
                for c in range(4):
                    pool_v[bi, pl.ds(c * 16, 16)] = acc[c]
            obase = pl.multiple_of(wid * B_PER_TILE + g * B_GROUP, B_GROUP)
            pltpu.sync_copy(pool_v, pooled_out.at[pl.ds(obase, B_GROUP)])
            return carry

        lax.fori_loop(0, N_B_GROUPS, ctx_group, 0)

    return k(item_table, ctx_table, item_idx_flat, ctx_idx_flat)


def _tc_score(item_embeds, pooled_raw, ctx_idx_pad, row0, W1, b1, W2, b2):
    BB = 512

    def body(item_ref, pool_ref, cidx_ref, row0_ref, W1_ref, b1_ref,
             W2_ref, b2_ref, out_ref):
        nz = jnp.sum((cidx_ref[...] == 0).astype(jnp.float32), axis=1,
                     keepdims=True)
        pooled = pool_ref[...] - nz * row0_ref[...]
        up = jnp.tanh(
            jnp.dot(pooled, W1_ref[...], preferred_element_type=jnp.float32)
            + b1_ref[...])
        ctx = (jnp.dot(up, W2_ref[...], preferred_element_type=jnp.float32)
               + b2_ref[...])
        diff = item_ref[...] - ctx[:, None, :]
        d2 = jnp.sum(diff * diff, axis=-1)
        out_ref[...] = 1.0 - jnp.tanh(jnp.sqrt(d2))

    return pl.pallas_call(
        body,
        grid=(B // BB,),
        in_specs=[
            pl.BlockSpec((BB, S, E), lambda i: (i, 0, 0)),
            pl.BlockSpec((BB, E), lambda i: (i, 0)),
            pl.BlockSpec((BB, S_PAD), lambda i: (i, 0)),
            pl.BlockSpec((1, E), lambda i: (0, 0)),
            pl.BlockSpec((E, 2 * E), lambda i: (0, 0)),
            pl.BlockSpec((1, 2 * E), lambda i: (0, 0)),
            pl.BlockSpec((2 * E, E), lambda i: (0, 0)),
            pl.BlockSpec((1, E), lambda i: (0, 0)),
        ],
        out_specs=pl.BlockSpec((BB, S), lambda i: (i, 0)),
        out_shape=jax.ShapeDtypeStruct((B, S), jnp.float32),
    )(item_embeds, pooled_raw, ctx_idx_pad, row0, W1, b1, W2, b2)


def kernel(item_indices, context_indices, item_table, ctx_table, W1, b1, W2, b2):
    iidx = item_indices.reshape(-1).astype(jnp.int32)
    cidx_pad = jnp.pad(context_indices.astype(jnp.int32),
                       ((0, 0), (0, S_PAD - S)))
    item_embeds, pooled_raw = _sc_gather_pool(
        item_table, ctx_table, iidx, cidx_pad.reshape(-1))
    row0 = lax.slice(ctx_table, (0, 0), (1, E))
    return _tc_score(item_embeds.reshape(B, S, E), pooled_raw, cidx_pad, row0,
                     W1, b1.reshape(1, -1), W2, b2.reshape(1, -1))


# speedup vs baseline: 1.2164x; 1.2164x over previous
import functools

import jax
import jax.numpy as jnp
from jax import lax
from jax.experimental import pallas as pl
from jax.experimental.pallas import tpu as pltpu
from jax.experimental.pallas import tpu_sc as plsc

NUM_ITEMS = 1000000
NPAIR = NUM_ITEMS // 2
B = 4096
S = 50
S_PAD = 56
E = 64
NW = 32

ITEMS_PER_TILE = (B * S) // NW
ITEM_CHUNK = 320
N_ITEM_CHUNKS = ITEMS_PER_TILE // ITEM_CHUNK
B_PER_TILE = B // NW
B_GROUP = 4
N_B_GROUPS = B_PER_TILE // B_GROUP
CTX_CHUNK = B_GROUP * S_PAD


def _sc_gather_pool(itab2, ctab2, ipair, cpair, coff, cmask):
    mesh = plsc.VectorSubcoreMesh(core_axis_name="c", subcore_axis_name="s")

    @functools.partial(
        pl.kernel,
        mesh=mesh,
        out_type=[
            jax.ShapeDtypeStruct((B * S, 2 * E), jnp.float32),
            jax.ShapeDtypeStruct((B, E), jnp.float32),
        ],
        scratch_types=[
            pltpu.VMEM((ITEM_CHUNK,), jnp.int32),
            pltpu.VMEM((ITEM_CHUNK,), jnp.int32),
            pltpu.VMEM((ITEM_CHUNK, 2 * E), jnp.float32),
            pltpu.VMEM((ITEM_CHUNK, 2 * E), jnp.float32),
            pltpu.VMEM((CTX_CHUNK,), jnp.int32),
            pltpu.VMEM((CTX_CHUNK + 16,), jnp.int32),
            pltpu.VMEM((CTX_CHUNK + 16,), jnp.float32),
            pltpu.VMEM((CTX_CHUNK, 2 * E), jnp.float32),
            pltpu.VMEM((B_GROUP, E), jnp.float32),
            pltpu.SemaphoreType.DMA,
            pltpu.SemaphoreType.DMA,
            pltpu.SemaphoreType.DMA,
            pltpu.SemaphoreType.DMA,
            pltpu.SemaphoreType.DMA,
        ],
    )
    def k(itab_hbm, ctab_hbm, ipair_hbm, cpair_hbm, coff_hbm, cmask_hbm,
          item_out, pooled_out,
          iidx0, iidx1, ibuf0, ibuf1, cidx_v, coff_v, cmask_v, cbuf, pool_v,
          sem_g0, sem_g1, sem_w0, sem_w1, sem_c):
        wid = lax.axis_index("s") * 2 + lax.axis_index("c")
        iidx = (iidx0, iidx1)
        ibuf = (ibuf0, ibuf1)
        sem_g = (sem_g0, sem_g1)
        sem_w = (sem_w0, sem_w1)

        def istart(kk):
            bsel = kk % 2
            base = wid * ITEMS_PER_TILE + kk * ITEM_CHUNK
            pltpu.sync_copy(ipair_hbm.at[pl.ds(base, ITEM_CHUNK)], iidx[bsel])
            return pltpu.async_copy(itab_hbm.at[iidx[bsel]], ibuf[bsel],
                                    sem_g[bsel])

        def iwrite(kk):
            bsel = kk % 2
            base = wid * ITEMS_PER_TILE + kk * ITEM_CHUNK
            return pltpu.async_copy(
                ibuf[bsel], item_out.at[pl.ds(base, ITEM_CHUNK)], sem_w[bsel])

        gathers = [istart(0)]
        writes = []
        for kk in range(1, N_ITEM_CHUNKS):
            bsel = kk % 2
            if kk >= 2:
                writes[kk - 2].wait()
            gathers.append(istart(kk))
            gathers[kk - 1].wait()
            writes.append(iwrite(kk - 1))
        gathers[N_ITEM_CHUNKS - 1].wait()
        writes.append(iwrite(N_ITEM_CHUNKS - 1))

        def ctx_group(g, carry):
            cbase = wid * (B_PER_TILE * S_PAD) + g * CTX_CHUNK
            pltpu.sync_copy(cpair_hbm.at[pl.ds(cbase, CTX_CHUNK)], cidx_v)
            pltpu.sync_copy(coff_hbm.at[pl.ds(cbase, CTX_CHUNK)],
                            coff_v.at[pl.ds(0, CTX_CHUNK)])
            pltpu.sync_copy(cmask_hbm.at[pl.ds(cbase, CTX_CHUNK)],
                            cmask_v.at[pl.ds(0, CTX_CHUNK)])
            pltpu.async_copy(ctab_hbm.at[cidx_v], cbuf, sem_c).wait()
            zero = jnp.zeros((16,), jnp.float32)
            for bi in range(B_GROUP):
                def body(s, acc, _bi=bi):
                    j = _bi * S_PAD + s
                    off = coff_v[pl.ds(j, 16)][0]
                    m = cmask_v[pl.ds(j, 16)][0]
                    return tuple(
                        acc[c] + m * cbuf[j, pl.ds(off + c * 16, 16)]
                        for c in range(4))

                acc = lax.fori_loop(0, S_PAD, body, (zero, zero, zero, zero))
                for c in range(4):
                    pool_v[bi, pl.ds(c * 16, 16)] = acc[c]
            obase = wid * B_PER_TILE + g * B_GROUP
            pltpu.sync_copy(pool_v, pooled_out.at[pl.ds(obase, B_GROUP)])
            return carry

        lax.fori_loop(0, N_B_GROUPS, ctx_group, 0)
        writes[N_ITEM_CHUNKS - 2].wait()
        writes[N_ITEM_CHUNKS - 1].wait()

    return k(itab2, ctab2, ipair, cpair, coff, cmask)


def _tc_score(item_pairs, iparity, pooled, W1, b1, W2, b2):
    BB = 256

    def body(item_ref, par_ref, pool_ref, W1_ref, b1_ref, W2_ref, b2_ref,
             out_ref):
        up = jnp.tanh(
            jnp.dot(pool_ref[...], W1_ref[...],
                    preferred_element_type=jnp.float32) + b1_ref[...])
        ctx = (jnp.dot(up, W2_ref[...], preferred_element_type=jnp.float32)
               + b2_ref[...])
        item3 = item_ref[...].reshape(BB, S, 2 * E)
        ctx128 = jnp.concatenate([ctx, ctx], axis=-1)
        diff = item3 - ctx128[:, None, :]
        sq = diff * diff
        lane = lax.broadcasted_iota(jnp.int32, (BB, S, 2 * E), 2)
        sel = (lane // E) == par_ref[...][:, :, None]
        d2 = jnp.sum(jnp.where(sel, sq, 0.0), axis=-1)
        out_ref[...] = 1.0 - jnp.tanh(jnp.sqrt(d2))

    return pl.pallas_call(
        body,
        grid=(B // BB,),
        in_specs=[
            pl.BlockSpec((BB * S, 2 * E), lambda i: (i, 0)),
            pl.BlockSpec((BB, S), lambda i: (i, 0)),
            pl.BlockSpec((BB, E), lambda i: (i, 0)),
            pl.BlockSpec((E, 2 * E), lambda i: (0, 0)),
            pl.BlockSpec((1, 2 * E), lambda i: (0, 0)),
            pl.BlockSpec((2 * E, E), lambda i: (0, 0)),
            pl.BlockSpec((1, E), lambda i: (0, 0)),
        ],
        out_specs=pl.BlockSpec((BB, S), lambda i: (i, 0)),
        out_shape=jax.ShapeDtypeStruct((B, S), jnp.float32),
    )(item_pairs, iparity, pooled, W1, b1, W2, b2)


def kernel(item_indices, context_indices, item_table, ctx_table, W1, b1, W2, b2):
    ii = item_indices.astype(jnp.int32)
    ci = context_indices.astype(jnp.int32)
    ipair = (ii >> 1).reshape(-1)
    iparity = ii & 1
    spread = (jnp.arange(B * (S_PAD - S), dtype=jnp.int32) * 7919) % NPAIR
    cpair = jnp.concatenate([ci >> 1, spread.reshape(B, S_PAD - S)], axis=1)
    coff = jnp.pad(ci & 1, ((0, 0), (0, S_PAD - S))) * E
    cmask = jnp.pad((ci > 0).astype(jnp.float32), ((0, 0), (0, S_PAD - S)))
    item_pairs, pooled = _sc_gather_pool(
        item_table.reshape(NPAIR, 2 * E), ctx_table.reshape(NPAIR, 2 * E),
        ipair, cpair.reshape(-1), coff.reshape(-1), cmask.reshape(-1))
    return _tc_score(item_pairs, iparity, pooled,
                     W1, b1.reshape(1, -1), W2, b2.reshape(1, -1))


# speedup vs baseline: 1.3113x; 1.0780x over previous
import functools

import jax
import jax.numpy as jnp
from jax import lax
from jax.experimental import pallas as pl
from jax.experimental.pallas import tpu as pltpu
from jax.experimental.pallas import tpu_sc as plsc

NUM_ITEMS = 1000000
B = 4096
S = 50
S_PAD = 56
E = 64
NW = 32

ITEMS_PER_TILE = (B * S) // NW
ITEM_CHUNK = 320
N_ITEM_CHUNKS = ITEMS_PER_TILE // ITEM_CHUNK
B_PER_TILE = B // NW
B_GROUP = 8
N_B_GROUPS = B_PER_TILE // B_GROUP
CTX_CHUNK = B_GROUP * S_PAD


def _sc_gather_pool(itab, ctab, iidx_flat, cidx_flat):
    mesh = plsc.VectorSubcoreMesh(core_axis_name="c", subcore_axis_name="s")

    @functools.partial(
        pl.kernel,
        mesh=mesh,
        out_type=[
            jax.ShapeDtypeStruct((B * S, E), jnp.float32),
            jax.ShapeDtypeStruct((B, E), jnp.float32),
        ],
        scratch_types=[
            pltpu.VMEM((ITEM_CHUNK,), jnp.int32),
            pltpu.VMEM((ITEM_CHUNK,), jnp.int32),
            pltpu.VMEM((ITEM_CHUNK, E), jnp.float32),
            pltpu.VMEM((ITEM_CHUNK, E), jnp.float32),
            pltpu.VMEM((CTX_CHUNK,), jnp.int32),
            pltpu.VMEM((CTX_CHUNK,), jnp.int32),
            pltpu.VMEM((CTX_CHUNK, E), jnp.float32),
            pltpu.VMEM((CTX_CHUNK, E), jnp.float32),
            pltpu.VMEM((B_GROUP, E), jnp.float32),
            pltpu.VMEM((B_GROUP, E), jnp.float32),
            pltpu.SemaphoreType.DMA,
            pltpu.SemaphoreType.DMA,
            pltpu.SemaphoreType.DMA,
            pltpu.SemaphoreType.DMA,
            pltpu.SemaphoreType.DMA,
            pltpu.SemaphoreType.DMA,
            pltpu.SemaphoreType.DMA,
            pltpu.SemaphoreType.DMA,
        ],
        compiler_params=pltpu.CompilerParams(use_tc_tiling_on_sc=False),
    )
    def k(itab_hbm, ctab_hbm, iidx_hbm, cidx_hbm,
          item_out, pooled_out,
          iidx0, iidx1, ibuf0, ibuf1, cidx0, cidx1, cbuf0, cbuf1,
          pool0, pool1,
          sem_ig0, sem_ig1, sem_iw0, sem_iw1, sem_cg0, sem_cg1,
          sem_pw0, sem_pw1):
        wid = lax.axis_index("s") * 2 + lax.axis_index("c")
        iidx = (iidx0, iidx1)
        ibuf = (ibuf0, ibuf1)
        cidx = (cidx0, cidx1)
        cbuf = (cbuf0, cbuf1)
        pool = (pool0, pool1)
        sem_ig = (sem_ig0, sem_ig1)
        sem_iw = (sem_iw0, sem_iw1)
        sem_cg = (sem_cg0, sem_cg1)
        sem_pw = (sem_pw0, sem_pw1)

        def istart(kk):
            bsel = kk % 2
            base = pl.multiple_of(wid * ITEMS_PER_TILE + kk * ITEM_CHUNK,
                                  ITEM_CHUNK)
            pltpu.sync_copy(iidx_hbm.at[pl.ds(base, ITEM_CHUNK)], iidx[bsel])
            return pltpu.async_copy(itab_hbm.at[iidx[bsel]], ibuf[bsel],
                                    sem_ig[bsel])

        def iwrite(kk):
            bsel = kk % 2
            base = pl.multiple_of(wid * ITEMS_PER_TILE + kk * ITEM_CHUNK,
                                  ITEM_CHUNK)
            return pltpu.async_copy(
                ibuf[bsel], item_out.at[pl.ds(base, ITEM_CHUNK)],
                sem_iw[bsel])

        def cstart(g):
            bsel = g % 2
            cbase = pl.multiple_of(
                wid * (B_PER_TILE * S_PAD) + g * CTX_CHUNK, CTX_CHUNK)
            pltpu.sync_copy(cidx_hbm.at[pl.ds(cbase, CTX_CHUNK)], cidx[bsel])
            return pltpu.async_copy(ctab_hbm.at[cidx[bsel]], cbuf[bsel],
                                    sem_cg[bsel])

        def item_step(t, gathers, writes):
            if t + 1 < N_ITEM_CHUNKS:
                if t >= 1:
                    writes[t - 1].wait()
                gathers.append(istart(t + 1))
            gathers[t].wait()
            writes.append(iwrite(t))

        gathers = [istart(0)]
        writes = []
        cgathers = [cstart(0)]
        pwrites = []

        for g in range(N_B_GROUPS):
            psel = g % 2
            bsel = g % 2
            cgathers[g].wait()
            if g + 1 < N_B_GROUPS:
                cgathers.append(cstart(g + 1))
            if g >= 2:
                pwrites[g - 2].wait()
            zero = jnp.zeros((16,), jnp.float32)
            for bi in range(B_GROUP):
                def body(s, acc, _bi=bi, _bsel=bsel):
                    j = _bi * S_PAD + s
                    return tuple(
                        acc[c] + cbuf[_bsel][j, pl.ds(c * 16, 16)]
                        for c in range(4))

                acc = lax.fori_loop(0, S, body, (zero, zero, zero, zero))
                for c in range(4):
                    pool[psel][bi, pl.ds(c * 16, 16)] = acc[c]
            obase = pl.multiple_of(wid * B_PER_TILE + g * B_GROUP, B_GROUP)
            pwrites.append(pltpu.async_copy(
                pool[psel], pooled_out.at[pl.ds(obase, B_GROUP)],
                sem_pw[psel]))
            for t in ([2 * g, 2 * g + 1] if g < 4 else [g + 4]):
                item_step(t, gathers, writes)

        pwrites[N_B_GROUPS - 2].wait()
        pwrites[N_B_GROUPS - 1].wait()
        writes[N_ITEM_CHUNKS - 2].wait()
        writes[N_ITEM_CHUNKS - 1].wait()

    return k(itab, ctab, iidx_flat, cidx_flat)


def _tc_score(item2, cidx, pooled, row0, W1, b1, W2, b2):
    BB = 256

    def body(item_ref, cidx_ref, pool_ref, row0_ref, W1_ref, b1_ref,
             W2_ref, b2_ref, out_ref):
        nz = jnp.sum((cidx_ref[...] == 0).astype(jnp.float32), axis=1,
                     keepdims=True)
        pooled_c = pool_ref[...] - nz * row0_ref[...]
        up = jnp.tanh(
            jnp.dot(pooled_c, W1_ref[...],
                    preferred_element_type=jnp.float32) + b1_ref[...])
        ctx = (jnp.dot(up, W2_ref[...], preferred_element_type=jnp.float32)
               + b2_ref[...])
        item3 = item_ref[...].reshape(BB, S, E)
        diff = item3 - ctx[:, None, :]
        d2 = jnp.sum(diff * diff, axis=-1)
        out_ref[...] = 1.0 - jnp.tanh(jnp.sqrt(d2))

    return pl.pallas_call(
        body,
        grid=(B // BB,),
        in_specs=[
            pl.BlockSpec((BB * S, E), lambda i: (i, 0)),
            pl.BlockSpec((BB, S), lambda i: (i, 0)),
            pl.BlockSpec((BB, E), lambda i: (i, 0)),
            pl.BlockSpec((1, E), lambda i: (0, 0)),
            pl.BlockSpec((E, 2 * E), lambda i: (0, 0)),
            pl.BlockSpec((1, 2 * E), lambda i: (0, 0)),
            pl.BlockSpec((2 * E, E), lambda i: (0, 0)),
            pl.BlockSpec((1, E), lambda i: (0, 0)),
        ],
        out_specs=pl.BlockSpec((BB, S), lambda i: (i, 0)),
        out_shape=jax.ShapeDtypeStruct((B, S), jnp.float32),
    )(item2, cidx, pooled, row0, W1, b1, W2, b2)


def kernel(item_indices, context_indices, item_table, ctx_table, W1, b1, W2, b2):
    ii = item_indices.astype(jnp.int32)
    ci = context_indices.astype(jnp.int32)
    spread = (jnp.arange(B * (S_PAD - S), dtype=jnp.int32) * 7919) % NUM_ITEMS
    cidx_pad = jnp.concatenate([ci, spread.reshape(B, S_PAD - S)], axis=1)
    item2, pooled = _sc_gather_pool(
        item_table, ctx_table, ii.reshape(-1), cidx_pad.reshape(-1))
    row0 = lax.slice(ctx_table, (0, 0), (1, E))
    return _tc_score(item2, ci, pooled, row0,
                     W1, b1.reshape(1, -1), W2, b2.reshape(1, -1))
